# baseline (speedup 1.0000x reference)
import jax
import jax.numpy as jnp
from jax.experimental import pallas as pl
from jax.experimental.pallas import tpu as pltpu

N = 16384
D = 256
K = 1024
BN = 2048
DA = D + 3


def _assign_kernel(x_ref, c_ref, out_ref, ca_ref, xa_ref):
    @pl.when(pl.program_id(0) == 0)
    def _():
        c = c_ref[...]
        cn = jnp.sum(c * c, axis=0, keepdims=True)
        cn1 = cn.astype(jnp.bfloat16).astype(jnp.float32)
        r1 = cn - cn1
        cn2 = r1.astype(jnp.bfloat16).astype(jnp.float32)
        cn3 = r1 - cn2
        ca_ref[pl.ds(0, D), :] = -2.0 * c
        ca_ref[pl.ds(D, 1), :] = cn1
        ca_ref[pl.ds(D + 1, 1), :] = cn2
        ca_ref[pl.ds(D + 2, 1), :] = cn3
        xa_ref[:, pl.ds(D, 3)] = jnp.ones((BN, 3), jnp.float32)

    xa_ref[:, pl.ds(0, D)] = x_ref[...]
    scores = jax.lax.dot_general(
        ca_ref[...], xa_ref[...],
        dimension_numbers=(((0,), (1,)), ((), ())),
        preferred_element_type=jnp.float32)
    am = jnp.argmin(scores, axis=0).astype(jnp.int32)
    out_ref[...] = am.reshape(1, 1, BN)


def kernel(x, centroids):
    out = pl.pallas_call(
        _assign_kernel,
        grid=(N // BN,),
        in_specs=[
            pl.BlockSpec((BN, D), lambda i: (i, 0)),
            pl.BlockSpec((D, K), lambda i: (0, 0)),
        ],
        out_specs=pl.BlockSpec((1, 1, BN), lambda i: (i, 0, 0)),
        out_shape=jax.ShapeDtypeStruct((N // BN, 1, BN), jnp.int32),
        scratch_shapes=[pltpu.VMEM((DA, K), jnp.float32),
                        pltpu.VMEM((BN, DA), jnp.float32)],
    )(x, centroids)
    return out.reshape(N)


# speedup vs baseline: 1.1682x; 1.1682x over previous
import jax
import jax.numpy as jnp
from jax.experimental import pallas as pl
from jax.experimental.pallas import tpu as pltpu

N = 16384
D = 256
K = 1024
BN = 8192
SB = 2048


def _assign_kernel(x_ref, c_ref, out_ref, cn_ref):
    @pl.when(pl.program_id(0) == 0)
    def _():
        c = c_ref[...]
        cn = jnp.sum(c * c, axis=0, keepdims=True)
        cn_ref[...] = cn.reshape(K, 1)

    for j in range(BN // SB):
        xj = x_ref[pl.ds(j * SB, SB), :]
        prod_t = jax.lax.dot_general(
            c_ref[...], xj,
            dimension_numbers=(((0,), (1,)), ((), ())),
            preferred_element_type=jnp.float32)
        scores = cn_ref[...] - 2.0 * prod_t
        am = jnp.argmin(scores, axis=0).astype(jnp.int32)
        out_ref[0, 0, pl.ds(j * SB, SB)] = am


def kernel(x, centroids):
    out = pl.pallas_call(
        _assign_kernel,
        grid=(N // BN,),
        in_specs=[
            pl.BlockSpec((BN, D), lambda i: (i, 0)),
            pl.BlockSpec((D, K), lambda i: (0, 0)),
        ],
        out_specs=pl.BlockSpec((1, 1, BN), lambda i: (i, 0, 0)),
        out_shape=jax.ShapeDtypeStruct((N // BN, 1, BN), jnp.int32),
        scratch_shapes=[pltpu.VMEM((K, 1), jnp.float32)],
    )(x, centroids)
    return out.reshape(N)


# speedup vs baseline: 1.2628x; 1.0810x over previous
import jax
import jax.numpy as jnp
from jax.experimental import pallas as pl
from jax.experimental.pallas import tpu as pltpu

N = 16384
D = 256
K = 1024
BN = 2048
SB = 2048


def _assign_kernel(x_ref, c_ref, out_ref, cn_ref):
    @pl.when(pl.program_id(0) == 0)
    def _():
        c = c_ref[...]
        cn = jnp.sum(c * c, axis=0, keepdims=True)
        cn_ref[...] = cn.reshape(K, 1)

    for j in range(BN // SB):
        xj = x_ref[pl.ds(j * SB, SB), :]
        prod_t = jax.lax.dot_general(
            c_ref[...], xj,
            dimension_numbers=(((0,), (1,)), ((), ())),
            preferred_element_type=jnp.float32)
        scores = cn_ref[...] - 2.0 * prod_t
        am = jnp.argmin(scores, axis=0).astype(jnp.int32)
        out_ref[0, 0, pl.ds(j * SB, SB)] = am


def kernel(x, centroids):
    out = pl.pallas_call(
        _assign_kernel,
        grid=(N // BN,),
        in_specs=[
            pl.BlockSpec((BN, D), lambda i: (i, 0)),
            pl.BlockSpec((D, K), lambda i: (0, 0)),
        ],
        out_specs=pl.BlockSpec((1, 1, BN), lambda i: (i, 0, 0)),
        out_shape=jax.ShapeDtypeStruct((N // BN, 1, BN), jnp.int32),
        scratch_shapes=[pltpu.VMEM((K, 1), jnp.float32)],
    )(x, centroids)
    return out.reshape(N)
